# Initial kernel scaffold; baseline (speedup 1.0000x reference)
import jax
import jax.numpy as jnp
from jax.experimental import pallas as pl


def kernel(actions, weight):
    raise NotImplementedError("write your pallas kernel here")


# speedup vs baseline: 4.9049x; 4.9049x over previous
import functools

import jax
import jax.numpy as jnp
from jax import lax
from jax.experimental import pallas as pl
from jax.experimental.pallas import tpu as pltpu
from jax.experimental.pallas import tpu_sc as plsc

_D = 32
_NC = 2
_NS = 16
_NW = _NC * _NS
_B = 16384 * 200
_BPW = _B // _NW
_C = 1600
_NCHUNK = _BPW // _C

_mesh = plsc.VectorSubcoreMesh(core_axis_name="c", subcore_axis_name="s")


@functools.partial(
    pl.kernel,
    mesh=_mesh,
    out_type=jax.ShapeDtypeStruct((_B, _D), jnp.float32),
    compiler_params=pltpu.CompilerParams(use_tc_tiling_on_sc=False),
    scratch_types=[
        pltpu.VMEM((_C,), jnp.int32),
        pltpu.VMEM((_C, _D), jnp.float32),
        pltpu.SemaphoreType.DMA,
    ],
)
def _embed_gather(idx_hbm, table_hbm, out_hbm, idx_v, rows_v, sem):
    wid = lax.axis_index("s") * _NC + lax.axis_index("c")
    base = wid * _BPW

    def body(j, carry):
        off = base + j * _C
        pltpu.sync_copy(idx_hbm.at[pl.ds(off, _C)], idx_v)
        pltpu.async_copy(table_hbm.at[idx_v], rows_v, sem).wait()
        pltpu.sync_copy(rows_v, out_hbm.at[pl.ds(off, _C)])
        return carry

    lax.fori_loop(0, _NCHUNK, body, 0)


def kernel(actions, weight):
    idx = actions.reshape(-1).astype(jnp.int32)
    out = _embed_gather(idx, weight)
    return out.reshape(actions.shape[0], actions.shape[1], _D)


# speedup vs baseline: 5.0394x; 1.0274x over previous
import functools

import jax
import jax.numpy as jnp
from jax import lax
from jax.experimental import pallas as pl
from jax.experimental.pallas import tpu as pltpu
from jax.experimental.pallas import tpu_sc as plsc

_D = 32
_NC = 2
_NS = 16
_NW = _NC * _NS
_B = 16384 * 200
_BPW = _B // _NW
_C = 1600
_NCHUNK = _BPW // _C

_mesh = plsc.VectorSubcoreMesh(core_axis_name="c", subcore_axis_name="s")


@functools.partial(
    pl.kernel,
    mesh=_mesh,
    out_type=jax.ShapeDtypeStruct((_B, _D), jnp.float32),
    compiler_params=pltpu.CompilerParams(use_tc_tiling_on_sc=False),
    scratch_types=[
        pltpu.VMEM((_C,), jnp.int32),
        pltpu.VMEM((_C,), jnp.int32),
        pltpu.VMEM((_C, _D), jnp.float32),
        pltpu.VMEM((_C, _D), jnp.float32),
        pltpu.SemaphoreType.DMA,
        pltpu.SemaphoreType.DMA,
        pltpu.SemaphoreType.DMA,
        pltpu.SemaphoreType.DMA,
        pltpu.SemaphoreType.DMA,
        pltpu.SemaphoreType.DMA,
    ],
)
def _embed_gather(idx_hbm, table_hbm, out_hbm, idx0, idx1, rows0, rows1,
                  isem0, isem1, gsem0, gsem1, osem0, osem1):
    wid = lax.axis_index("s") * _NC + lax.axis_index("c")
    base = wid * _BPW

    idx_v = (idx0, idx1)
    rows_v = (rows0, rows1)
    isem = (isem0, isem1)
    gsem = (gsem0, gsem1)
    osem = (osem0, osem1)

    def idx_start(j, b):
        pltpu.async_copy(idx_hbm.at[pl.ds(base + j * _C, _C)], idx_v[b],
                         isem[b])

    def idx_wait(b):
        pltpu.make_async_copy(idx_hbm.at[pl.ds(base, _C)], idx_v[b],
                              isem[b]).wait()

    def gather_start(b):
        pltpu.async_copy(table_hbm.at[idx_v[b]], rows_v[b], gsem[b])

    def gather_wait(b):
        pltpu.make_async_copy(table_hbm.at[idx_v[b]], rows_v[b],
                              gsem[b]).wait()

    def store_start(j, b):
        pltpu.async_copy(rows_v[b], out_hbm.at[pl.ds(base + j * _C, _C)],
                         osem[b])

    def store_wait(b):
        pltpu.make_async_copy(rows_v[b], out_hbm.at[pl.ds(base, _C)],
                              osem[b]).wait()

    idx_start(0, 0)
    idx_start(1, 1)
    idx_wait(0)
    gather_start(0)

    gather_wait(0)
    idx_start(2, 0)
    idx_wait(1)
    gather_start(1)
    store_start(0, 0)

    gather_wait(1)
    idx_start(3, 1)
    store_wait(0)
    idx_wait(0)
    gather_start(0)
    store_start(1, 1)

    def body(jj, carry):
        j0 = 2 * jj
        gather_wait(0)
        idx_start(j0 + 2, 0)
        store_wait(1)
        idx_wait(1)
        gather_start(1)
        store_start(j0, 0)

        gather_wait(1)
        idx_start(j0 + 3, 1)
        store_wait(0)
        idx_wait(0)
        gather_start(0)
        store_start(j0 + 1, 1)
        return carry

    lax.fori_loop(1, _NCHUNK // 2 - 1, body, 0)

    gather_wait(0)
    store_wait(1)
    idx_wait(1)
    gather_start(1)
    store_start(_NCHUNK - 2, 0)

    gather_wait(1)
    store_start(_NCHUNK - 1, 1)
    store_wait(0)
    store_wait(1)


def kernel(actions, weight):
    idx = actions.reshape(-1).astype(jnp.int32)
    out = _embed_gather(idx, weight)
    return out.reshape(actions.shape[0], actions.shape[1], _D)


# speedup vs baseline: 5.0534x; 1.0028x over previous
import functools

import jax
import jax.numpy as jnp
from jax import lax
from jax.experimental import pallas as pl
from jax.experimental.pallas import tpu as pltpu
from jax.experimental.pallas import tpu_sc as plsc

_D = 32
_NC = 2
_NS = 16
_NW = _NC * _NS
_B = 16384 * 200
_BPW = _B // _NW
_C = 800
_NCHUNK = _BPW // _C
_NBUF = 4

_mesh = plsc.VectorSubcoreMesh(core_axis_name="c", subcore_axis_name="s")


@functools.partial(
    pl.kernel,
    mesh=_mesh,
    out_type=jax.ShapeDtypeStruct((_B, _D), jnp.float32),
    compiler_params=pltpu.CompilerParams(use_tc_tiling_on_sc=False),
    scratch_types=(
        [pltpu.VMEM((_C,), jnp.int32) for _ in range(_NBUF)]
        + [pltpu.VMEM((_C, _D), jnp.float32) for _ in range(_NBUF)]
        + [pltpu.SemaphoreType.DMA for _ in range(3 * _NBUF)]
    ),
)
def _embed_gather(idx_hbm, table_hbm, out_hbm, *scratch):
    idx_v = scratch[0:_NBUF]
    rows_v = scratch[_NBUF:2 * _NBUF]
    isem = scratch[2 * _NBUF:3 * _NBUF]
    gsem = scratch[3 * _NBUF:4 * _NBUF]
    osem = scratch[4 * _NBUF:5 * _NBUF]

    wid = lax.axis_index("s") * _NC + lax.axis_index("c")
    base = wid * _BPW

    def idx_start(j, b):
        pltpu.async_copy(idx_hbm.at[pl.ds(base + j * _C, _C)], idx_v[b],
                         isem[b])

    def idx_wait(b):
        pltpu.make_async_copy(idx_hbm.at[pl.ds(base, _C)], idx_v[b],
                              isem[b]).wait()

    def gather_start(b):
        pltpu.async_copy(table_hbm.at[idx_v[b]], rows_v[b], gsem[b])

    def gather_wait(b):
        pltpu.make_async_copy(table_hbm.at[idx_v[b]], rows_v[b],
                              gsem[b]).wait()

    def store_start(j, b):
        pltpu.async_copy(rows_v[b], out_hbm.at[pl.ds(base + j * _C, _C)],
                         osem[b])

    def store_wait(b):
        pltpu.make_async_copy(rows_v[b], out_hbm.at[pl.ds(base, _C)],
                              osem[b]).wait()

    def stage(j, b, do_idx, do_gather, first):
        b2 = (b - 1) % _NBUF
        gather_wait(b)
        if do_idx:
            idx_start(j + _NBUF, b)
        store_start(j, b)
        if do_gather:
            if not first:
                store_wait(b2)
            idx_wait(b2)
            gather_start(b2)

    for k in range(_NBUF):
        idx_start(k, k)
    for k in range(_NBUF - 1):
        idx_wait(k)
        gather_start(k)

    stage(0, 0, True, True, True)
    for k in range(1, _NBUF):
        stage(k, k, True, True, False)

    def body(jj, carry):
        j0 = _NBUF * jj
        for k in range(_NBUF):
            stage(j0 + k, k, True, True, False)
        return carry

    lax.fori_loop(1, _NCHUNK // _NBUF - 1, body, 0)

    stage(_NCHUNK - _NBUF, 0, False, True, False)
    for k in range(1, _NBUF):
        stage(_NCHUNK - _NBUF + k, k, False, False, False)

    for k in range(_NBUF):
        store_wait(k)


def kernel(actions, weight):
    idx = actions.reshape(-1).astype(jnp.int32)
    out = _embed_gather(idx, weight)
    return out.reshape(actions.shape[0], actions.shape[1], _D)
